# Initial kernel scaffold; baseline (speedup 1.0000x reference)
import jax
import jax.numpy as jnp
from jax.experimental import pallas as pl


def kernel(indices, offsets, weights):
    raise NotImplementedError("write your pallas kernel here")


# speedup vs baseline: 173.7642x; 173.7642x over previous
import functools

import jax
import jax.numpy as jnp
from jax import lax
from jax.experimental import pallas as pl
from jax.experimental.pallas import tpu as pltpu
from jax.experimental.pallas import tpu_sc as plsc

T = 26
B = 4096
L = 20
VOCAB = 100000
D = 64

NC = 2
NS = 16
NW = NC * NS

CHUNK = 32
ROWS = CHUNK * L
NGATHER = ROWS // 128
CHUNKS_PER_TABLE = B // CHUNK
TOTAL_CHUNKS = T * CHUNKS_PER_TABLE
CPW = TOTAL_CHUNKS // NW


def _tec_body(idx_hbm, w_hbm, out_hbm, idx_raw, idx_adj, rows, pooled, sem):
    wid = lax.axis_index("s") * NC + lax.axis_index("c")
    base_chunk = wid * CPW

    def chunk_body(i, carry):
        c = base_chunk + i
        t = c // CHUNKS_PER_TABLE
        b0 = (c % CHUNKS_PER_TABLE) * CHUNK
        tbase = t * VOCAB

        pltpu.sync_copy(idx_hbm.at[pl.ds(c * ROWS, ROWS)], idx_raw)
        for j in range(ROWS // 16):
            v = idx_raw[pl.ds(j * 16, 16)] + tbase
            idx_adj[j // 8, pl.ds((j % 8) * 16, 16)] = v

        copies = [
            pltpu.async_copy(
                w_hbm.at[idx_adj.at[g]], rows.at[pl.ds(g * 128, 128)], sem
            )
            for g in range(NGATHER)
        ]
        for cp in copies:
            cp.wait()

        def bag_body(k, carry2):
            r0 = k * L
            acc = [rows[r0, pl.ds(cc * 16, 16)] for cc in range(D // 16)]
            for l in range(1, L):
                for cc in range(D // 16):
                    acc[cc] = acc[cc] + rows[r0 + l, pl.ds(cc * 16, 16)]
            for cc in range(D // 16):
                pooled[k, pl.ds(cc * 16, 16)] = acc[cc]
            return carry2

        lax.fori_loop(0, CHUNK, bag_body, 0, unroll=1)

        pltpu.sync_copy(
            pooled, out_hbm.at[pl.ds(b0, CHUNK), pl.ds(t * D, D)]
        )
        return carry

    lax.fori_loop(0, CPW, chunk_body, 0, unroll=1)


def kernel(indices, offsets, weights):
    del offsets
    mesh = plsc.VectorSubcoreMesh(core_axis_name="c", subcore_axis_name="s")
    k = functools.partial(
        pl.kernel,
        mesh=mesh,
        compiler_params=pltpu.CompilerParams(use_tc_tiling_on_sc=False),
        out_type=jax.ShapeDtypeStruct((B, T * D), jnp.float32),
        scratch_types=[
            pltpu.VMEM((ROWS,), jnp.int32),
            pltpu.VMEM((NGATHER, 128), jnp.int32),
            pltpu.VMEM((ROWS, D), jnp.float32),
            pltpu.VMEM((CHUNK, D), jnp.float32),
            pltpu.SemaphoreType.DMA,
        ],
    )(_tec_body)
    return k(indices, weights)


# speedup vs baseline: 191.8796x; 1.1043x over previous
import functools

import jax
import jax.numpy as jnp
from jax import lax
from jax.experimental import pallas as pl
from jax.experimental.pallas import tpu as pltpu
from jax.experimental.pallas import tpu_sc as plsc

T = 26
B = 4096
L = 20
VOCAB = 100000
D = 64

NC = 2
NS = 16
NW = NC * NS

CHUNK = 32
ROWS = CHUNK * L
NGATHER = ROWS // 128
CHUNKS_PER_TABLE = B // CHUNK
TOTAL_CHUNKS = T * CHUNKS_PER_TABLE
CPW = TOTAL_CHUNKS // NW
SUPER = 8
NSUPER = CPW // SUPER
SROWS = SUPER * ROWS
SBAGS = SUPER * CHUNK


def _tec_body(idx_hbm, w_hbm, out_hbm, idxr, idxa, rows, pooled, sg0, sg1):
    wid = lax.axis_index("s") * NC + lax.axis_index("c")
    base = wid * CPW
    sems = (sg0, sg1)

    def adjust_and_fire(k, tbase, kb):
        off = k * ROWS

        def adj_body(g, _):
            for jj in range(8):
                v = idxr[pl.ds(off + g * 128 + jj * 16, 16)] + tbase
                idxa[kb, g, pl.ds(jj * 16, 16)] = v
            return 0

        lax.fori_loop(0, NGATHER, adj_body, 0, unroll=1)
        return [
            pltpu.async_copy(
                w_hbm.at[idxa.at[kb, g]],
                rows.at[kb, pl.ds(g * 128, 128)],
                sems[kb],
            )
            for g in range(NGATHER)
        ]

    def accumulate(k, kb):
        p0 = k * CHUNK

        def bag_body(bb, _):
            r0 = bb * L
            acc = [rows[kb, r0, pl.ds(cc * 16, 16)] for cc in range(D // 16)]
            for l in range(1, L):
                for cc in range(D // 16):
                    acc[cc] = acc[cc] + rows[kb, r0 + l, pl.ds(cc * 16, 16)]
            for cc in range(D // 16):
                pooled[p0 + bb, pl.ds(cc * 16, 16)] = acc[cc]
            return 0

        lax.fori_loop(0, CHUNK, bag_body, 0, unroll=2)

    def super_body(s, _):
        c0 = base + s * SUPER
        g0 = c0 * CHUNK
        t = c0 // CHUNKS_PER_TABLE
        b0 = g0 - t * B
        tbase = t * VOCAB

        pltpu.sync_copy(idx_hbm.at[pl.ds(c0 * ROWS, SROWS)], idxr)
        cps = adjust_and_fire(0, tbase, 0)
        for k in range(SUPER):
            nxt = adjust_and_fire(k + 1, tbase, (k + 1) % 2) if k + 1 < SUPER else None
            for cp in cps:
                cp.wait()
            accumulate(k, k % 2)
            cps = nxt

        pltpu.sync_copy(pooled, out_hbm.at[pl.ds(b0, SBAGS), pl.ds(t * D, D)])
        return 0

    lax.fori_loop(0, NSUPER, super_body, 0, unroll=1)


def kernel(indices, offsets, weights):
    del offsets
    mesh = plsc.VectorSubcoreMesh(core_axis_name="c", subcore_axis_name="s")
    k = functools.partial(
        pl.kernel,
        mesh=mesh,
        compiler_params=pltpu.CompilerParams(use_tc_tiling_on_sc=False),
        out_type=jax.ShapeDtypeStruct((B, T * D), jnp.float32),
        scratch_types=[
            pltpu.VMEM((SROWS,), jnp.int32),
            pltpu.VMEM((2, NGATHER, 128), jnp.int32),
            pltpu.VMEM((2, ROWS, D), jnp.float32),
            pltpu.VMEM((SBAGS, D), jnp.float32),
            pltpu.SemaphoreType.DMA,
            pltpu.SemaphoreType.DMA,
        ],
    )(_tec_body)
    return k(indices, weights)
